# baseline (speedup 1.0000x reference)
import functools

import jax
import jax.numpy as jnp
from jax import lax
from jax.experimental import pallas as pl
from jax.experimental.pallas import tpu as pltpu
from jax.experimental.pallas import tpu_sc as plsc

N = 10000
NP = 10240
D_IN = 128
DH = 128
DO = 64
E = 320000
NC, NS = 2, 16
NT = NC * NS
CH = 128
NCHUNK = 80
EPT = CH * NCHUNK
EP = EPT * NT
RPS = NP // NS
DEGW = 16

_mesh = functools.partial(
    plsc.VectorSubcoreMesh,
    core_axis_name="c", subcore_axis_name="s", num_cores=NC, num_subcores=NS,
)

_sc_params = pltpu.CompilerParams(use_tc_tiling_on_sc=False)


def _zero_fill(buf, f):
    @pl.loop(0, 64)
    def _(i):
        for k in range(f // 16):
            buf[i, pl.ds(16 * k, 16)] = jnp.zeros((16,), jnp.float32)


def _zero_shared(acc_sh, zbuf, sid):
    @pl.loop(0, RPS // 64)
    def _(j):
        pltpu.sync_copy(zbuf, acc_sh.at[pl.ds(sid * RPS + j * 64, 64)])


@functools.partial(
    pl.kernel,
    out_type=jax.ShapeDtypeStruct((NC, NP, DEGW), jnp.float32),
    mesh=_mesh(),
    compiler_params=_sc_params,
    scratch_types=[
        pltpu.VMEM((2, CH), jnp.int32),
        pltpu.VMEM((CH, DEGW), jnp.float32),
        pltpu.VMEM((64, DEGW), jnp.float32),
        pltpu.VMEM_SHARED((NP, DEGW), jnp.float32),
    ],
)
def _deg_pass(e_hbm, out_hbm, idxv, ones_v, zbuf, acc_sh):
    cid = lax.axis_index("c")
    sid = lax.axis_index("s")
    wid = cid * NS + sid

    @pl.loop(0, CH)
    def _(i):
        ones_v[i, :] = jnp.ones((16,), jnp.float32)

    _zero_fill(zbuf, DEGW)
    _zero_shared(acc_sh, zbuf, sid)
    plsc.subcore_barrier()

    base = wid * NCHUNK

    @pl.loop(0, NCHUNK)
    def _(j):
        pltpu.sync_copy(e_hbm.at[base + j], idxv)
        pltpu.sync_copy(ones_v, acc_sh.at[idxv.at[1]], add=True)

    plsc.subcore_barrier()
    pltpu.sync_copy(acc_sh.at[pl.ds(sid * RPS, RPS)],
                    out_hbm.at[cid, pl.ds(sid * RPS, RPS)])


def _make_edge_pass(f):
    @functools.partial(
        pl.kernel,
        out_type=jax.ShapeDtypeStruct((NC, NP, f), jnp.float32),
        mesh=_mesh(),
        compiler_params=_sc_params,
        scratch_types=[
            pltpu.VMEM((2, CH), jnp.int32),
            pltpu.VMEM((2, CH), jnp.int32),
            pltpu.VMEM((CH, f), jnp.float32),
            pltpu.VMEM((CH, f), jnp.float32),
            pltpu.VMEM((64, f), jnp.float32),
            pltpu.VMEM_SHARED((NP, f), jnp.float32),
            pltpu.SemaphoreType.DMA,
            pltpu.SemaphoreType.DMA,
        ],
    )
    def edge_pass(hs_hbm, e_hbm, out_hbm,
                  idxa, idxb, rowsa, rowsb, zbuf, acc_sh, gsa, gsb):
        cid = lax.axis_index("c")
        sid = lax.axis_index("s")
        wid = cid * NS + sid

        _zero_fill(zbuf, f)
        _zero_shared(acc_sh, zbuf, sid)
        plsc.subcore_barrier()

        base = wid * NCHUNK

        def wait_gather(rows, sem):
            pltpu.make_async_copy(hs_hbm.at[pl.ds(0, CH)], rows, sem).wait()

        pltpu.sync_copy(e_hbm.at[base], idxa)
        pltpu.async_copy(hs_hbm.at[idxa.at[0]], rowsa, gsa)

        npairs = NCHUNK // 2

        @pl.loop(0, npairs)
        def _(p):
            c1 = base + 2 * p + 1
            pltpu.sync_copy(e_hbm.at[c1], idxb)
            pltpu.async_copy(hs_hbm.at[idxb.at[0]], rowsb, gsb)
            wait_gather(rowsa, gsa)
            pltpu.sync_copy(rowsa, acc_sh.at[idxa.at[1]], add=True)

            @pl.when(p < npairs - 1)
            def _():
                pltpu.sync_copy(e_hbm.at[c1 + 1], idxa)
                pltpu.async_copy(hs_hbm.at[idxa.at[0]], rowsa, gsa)

            wait_gather(rowsb, gsb)
            pltpu.sync_copy(rowsb, acc_sh.at[idxb.at[1]], add=True)

        plsc.subcore_barrier()
        pltpu.sync_copy(acc_sh.at[pl.ds(sid * RPS, RPS)],
                        out_hbm.at[cid, pl.ds(sid * RPS, RPS)])

    return edge_pass


_edge_pass_h = _make_edge_pass(DH)
_edge_pass_o = _make_edge_pass(DO)


def _dis_from_deg(deg_ref):
    deg = deg_ref[0, :, 0:1] + deg_ref[1, :, 0:1]
    rows = lax.broadcasted_iota(jnp.int32, (NP, 1), 0)
    deg = deg + jnp.where(rows < N, 1.0, 0.0)
    return jnp.where(deg > 0, lax.rsqrt(deg), 0.0)


def _tc1_body(x_ref, w_ref, deg_ref, o_ref):
    dis = _dis_from_deg(deg_ref)
    h = jnp.dot(x_ref[...], w_ref[...], preferred_element_type=jnp.float32)
    o_ref[...] = h * dis


def _tc2_body(acc_ref, hs1_ref, deg_ref, b1_ref, g_ref, be_ref, w_ref, o_ref):
    dis = _dis_from_deg(deg_ref)
    t = (acc_ref[0] + acc_ref[1] + hs1_ref[...]) * dis + b1_ref[...]
    t = jnp.maximum(t, 0.0)
    mu = jnp.mean(t, axis=-1, keepdims=True)
    var = jnp.mean((t - mu) ** 2, axis=-1, keepdims=True)
    h = (t - mu) * lax.rsqrt(var + 1e-5) * g_ref[...] + be_ref[...]
    o_ref[...] = jnp.dot(h, w_ref[...],
                         preferred_element_type=jnp.float32) * dis


def _tc3_body(acc_ref, hs2_ref, deg_ref, b2_ref, o_ref):
    dis = _dis_from_deg(deg_ref)
    res = (acc_ref[0] + acc_ref[1] + hs2_ref[...]) * dis + b2_ref[...]
    o_ref[...] = res[:N]


_tc1 = pl.pallas_call(_tc1_body,
                      out_shape=jax.ShapeDtypeStruct((NP, DH), jnp.float32))
_tc2 = pl.pallas_call(_tc2_body,
                      out_shape=jax.ShapeDtypeStruct((NP, DO), jnp.float32))
_tc3 = pl.pallas_call(_tc3_body,
                      out_shape=jax.ShapeDtypeStruct((N, DO), jnp.float32))


def kernel(x, edge_index, W1, b1, gamma, beta, W2, b2):
    ei = edge_index.astype(jnp.int32)
    pad = jnp.full((2, EP - E), N, jnp.int32)
    e3 = jnp.concatenate([ei, pad], axis=1)
    e3 = e3.reshape(2, NT * NCHUNK, CH).transpose(1, 0, 2)
    x_pad = jnp.zeros((NP, D_IN), jnp.float32).at[:N].set(x)

    degp = _deg_pass(e3)
    hs1 = _tc1(x_pad, W1, degp)
    acc1 = _edge_pass_h(hs1, e3)
    hs2 = _tc2(acc1, hs1, degp, b1.reshape(1, DH),
               gamma.reshape(1, DH), beta.reshape(1, DH), W2)
    acc2 = _edge_pass_o(hs2, e3)
    return _tc3(acc2, hs2, degp, b2.reshape(1, DO))


# speedup vs baseline: 1.8865x; 1.8865x over previous
import functools

import jax
import jax.numpy as jnp
from jax import lax
from jax.experimental import pallas as pl
from jax.experimental.pallas import tpu as pltpu
from jax.experimental.pallas import tpu_sc as plsc

N = 10000
NP = 10240
D_IN = 128
DH = 128
DO = 64
E = 320000
NC, NS = 2, 16
NT = NC * NS
CH = 128
NCHUNK = 80
NCHUNK2 = 160
EPT = CH * NCHUNK
EP = EPT * NT
RPS = NP // NS
DEGW = 16

_mesh = functools.partial(
    plsc.VectorSubcoreMesh,
    core_axis_name="c", subcore_axis_name="s", num_cores=NC, num_subcores=NS,
)

_sc_params = pltpu.CompilerParams(use_tc_tiling_on_sc=False)


def _zero_fill(buf, f):
    @pl.loop(0, 64)
    def _(i):
        for k in range(f // 16):
            buf[i, pl.ds(16 * k, 16)] = jnp.zeros((16,), jnp.float32)


def _zero_shared(acc_sh, zbuf, sid):
    @pl.loop(0, RPS // 64)
    def _(j):
        pltpu.sync_copy(zbuf, acc_sh.at[pl.ds(sid * RPS + j * 64, 64)])


@functools.partial(
    pl.kernel,
    out_type=jax.ShapeDtypeStruct((NC, NP, DEGW), jnp.float32),
    mesh=_mesh(),
    compiler_params=_sc_params,
    scratch_types=[
        pltpu.VMEM((2, CH), jnp.int32),
        pltpu.VMEM((CH, DEGW), jnp.float32),
        pltpu.VMEM((64, DEGW), jnp.float32),
        pltpu.VMEM_SHARED((NP, DEGW), jnp.float32),
    ],
)
def _deg_pass(e_hbm, out_hbm, idxv, ones_v, zbuf, acc_sh):
    cid = lax.axis_index("c")
    sid = lax.axis_index("s")
    wid = cid * NS + sid

    @pl.loop(0, CH)
    def _(i):
        ones_v[i, :] = jnp.ones((16,), jnp.float32)

    _zero_fill(zbuf, DEGW)
    _zero_shared(acc_sh, zbuf, sid)
    plsc.subcore_barrier()

    base = wid * NCHUNK

    @pl.loop(0, NCHUNK)
    def _(j):
        pltpu.sync_copy(e_hbm.at[base + j], idxv)
        pltpu.sync_copy(ones_v, acc_sh.at[idxv.at[1]], add=True)

    plsc.subcore_barrier()
    pltpu.sync_copy(acc_sh.at[pl.ds(sid * RPS, RPS)],
                    out_hbm.at[cid, pl.ds(sid * RPS, RPS)])


def _make_edge_pass(f):
    h = f // 2

    @functools.partial(
        pl.kernel,
        out_type=jax.ShapeDtypeStruct((NC, NP, h), jnp.float32),
        mesh=_mesh(),
        compiler_params=_sc_params,
        scratch_types=[
            pltpu.VMEM((2, CH), jnp.int32),
            pltpu.VMEM((2, CH), jnp.int32),
            pltpu.VMEM((CH, h), jnp.float32),
            pltpu.VMEM((CH, h), jnp.float32),
            pltpu.VMEM((64, h), jnp.float32),
            pltpu.VMEM_SHARED((NP, h), jnp.float32),
            pltpu.VMEM_SHARED((NP, h), jnp.float32),
            pltpu.SemaphoreType.DMA,
            pltpu.SemaphoreType.DMA,
        ],
    )
    def edge_pass(hs_hbm, e_hbm, out_hbm,
                  idxa, idxb, rowsa, rowsb, zbuf, tab_sh, acc_sh, gsa, gsb):
        cid = lax.axis_index("c")
        sid = lax.axis_index("s")

        pltpu.sync_copy(hs_hbm.at[cid, pl.ds(sid * RPS, RPS)],
                        tab_sh.at[pl.ds(sid * RPS, RPS)])
        _zero_fill(zbuf, h)
        _zero_shared(acc_sh, zbuf, sid)
        plsc.subcore_barrier()

        base = sid * NCHUNK2

        def wait_gather(rows, sem):
            pltpu.make_async_copy(hs_hbm.at[0, pl.ds(0, CH)], rows, sem).wait()

        pltpu.sync_copy(e_hbm.at[base], idxa)
        pltpu.async_copy(tab_sh.at[idxa.at[0]], rowsa, gsa)

        npairs = NCHUNK2 // 2

        @pl.loop(0, npairs)
        def _(p):
            c1 = base + 2 * p + 1
            pltpu.sync_copy(e_hbm.at[c1], idxb)
            pltpu.async_copy(tab_sh.at[idxb.at[0]], rowsb, gsb)
            wait_gather(rowsa, gsa)
            pltpu.sync_copy(rowsa, acc_sh.at[idxa.at[1]], add=True)

            @pl.when(p < npairs - 1)
            def _():
                pltpu.sync_copy(e_hbm.at[c1 + 1], idxa)
                pltpu.async_copy(tab_sh.at[idxa.at[0]], rowsa, gsa)

            wait_gather(rowsb, gsb)
            pltpu.sync_copy(rowsb, acc_sh.at[idxb.at[1]], add=True)

        plsc.subcore_barrier()
        pltpu.sync_copy(acc_sh.at[pl.ds(sid * RPS, RPS)],
                        out_hbm.at[cid, pl.ds(sid * RPS, RPS)])

    return edge_pass


_edge_pass_h = _make_edge_pass(DH)
_edge_pass_o = _make_edge_pass(DO)


def _dis_from_deg(deg_ref):
    deg = deg_ref[0, :, 0:1] + deg_ref[1, :, 0:1]
    rows = lax.broadcasted_iota(jnp.int32, (NP, 1), 0)
    deg = deg + jnp.where(rows < N, 1.0, 0.0)
    return jnp.where(deg > 0, lax.rsqrt(deg), 0.0)


def _tc1_body(x_ref, w_ref, deg_ref, o_ref):
    dis = _dis_from_deg(deg_ref)
    hh = jnp.dot(x_ref[...], w_ref[...],
                 preferred_element_type=jnp.float32) * dis
    o_ref[0] = hh[:, :DH // 2]
    o_ref[1] = hh[:, DH // 2:]


def _tc2_body(acc_ref, hs1_ref, deg_ref, b1_ref, g_ref, be_ref, w_ref, o_ref):
    dis = _dis_from_deg(deg_ref)
    acc = jnp.concatenate([acc_ref[0] + hs1_ref[0],
                           acc_ref[1] + hs1_ref[1]], axis=1)
    t = acc * dis + b1_ref[...]
    t = jnp.maximum(t, 0.0)
    mu = jnp.mean(t, axis=-1, keepdims=True)
    var = jnp.mean((t - mu) ** 2, axis=-1, keepdims=True)
    h = (t - mu) * lax.rsqrt(var + 1e-5) * g_ref[...] + be_ref[...]
    hs2 = jnp.dot(h, w_ref[...], preferred_element_type=jnp.float32) * dis
    o_ref[0] = hs2[:, :DO // 2]
    o_ref[1] = hs2[:, DO // 2:]


def _tc3_body(acc_ref, hs2_ref, deg_ref, b2_ref, o_ref):
    dis = _dis_from_deg(deg_ref)
    res = jnp.concatenate([acc_ref[0] + hs2_ref[0],
                           acc_ref[1] + hs2_ref[1]], axis=1)
    o_ref[...] = (res * dis + b2_ref[...])[:N]


_tc1 = pl.pallas_call(_tc1_body,
                      out_shape=jax.ShapeDtypeStruct((NC, NP, DH // 2),
                                                     jnp.float32))
_tc2 = pl.pallas_call(_tc2_body,
                      out_shape=jax.ShapeDtypeStruct((NC, NP, DO // 2),
                                                     jnp.float32))
_tc3 = pl.pallas_call(_tc3_body,
                      out_shape=jax.ShapeDtypeStruct((N, DO), jnp.float32))


def kernel(x, edge_index, W1, b1, gamma, beta, W2, b2):
    ei = edge_index.astype(jnp.int32)
    pad = jnp.full((2, EP - E), N, jnp.int32)
    e3 = jnp.concatenate([ei, pad], axis=1)
    e3 = e3.reshape(2, NT * NCHUNK, CH).transpose(1, 0, 2)
    x_pad = jnp.zeros((NP, D_IN), jnp.float32).at[:N].set(x)

    degp = _deg_pass(e3)
    hs1 = _tc1(x_pad, W1, degp)
    acc1 = _edge_pass_h(hs1, e3)
    hs2 = _tc2(acc1, hs1, degp, b1.reshape(1, DH),
               gamma.reshape(1, DH), beta.reshape(1, DH), W2)
    acc2 = _edge_pass_o(hs2, e3)
    return _tc3(acc2, hs2, degp, b2.reshape(1, DO))


# speedup vs baseline: 2.1902x; 1.1610x over previous
import functools

import jax
import jax.numpy as jnp
from jax import lax
from jax.experimental import pallas as pl
from jax.experimental.pallas import tpu as pltpu
from jax.experimental.pallas import tpu_sc as plsc

N = 10000
NP = 10240
D_IN = 128
DH = 128
DO = 64
E = 320000
NC, NS = 2, 16
NT = NC * NS
CH = 128
NCHUNK = 80
NCHUNK2 = 160
EPT = CH * NCHUNK
EP = EPT * NT
RPS = NP // NS
DEGW = 16

_mesh = functools.partial(
    plsc.VectorSubcoreMesh,
    core_axis_name="c", subcore_axis_name="s", num_cores=NC, num_subcores=NS,
)

_sc_params = pltpu.CompilerParams(use_tc_tiling_on_sc=False)


def _zero_fill(buf, f):
    @pl.loop(0, 64)
    def _(i):
        for k in range(f // 16):
            buf[i, pl.ds(16 * k, 16)] = jnp.zeros((16,), jnp.float32)


def _zero_shared(acc_sh, zbuf, sid):
    @pl.loop(0, RPS // 64)
    def _(j):
        pltpu.sync_copy(zbuf, acc_sh.at[pl.ds(sid * RPS + j * 64, 64)])


@functools.partial(
    pl.kernel,
    out_type=jax.ShapeDtypeStruct((NC, NP, DEGW), jnp.float32),
    mesh=_mesh(),
    compiler_params=_sc_params,
    scratch_types=[
        pltpu.VMEM((2, CH), jnp.int32),
        pltpu.VMEM((CH, DEGW), jnp.float32),
        pltpu.VMEM((64, DEGW), jnp.float32),
        pltpu.VMEM_SHARED((NP, DEGW), jnp.float32),
    ],
)
def _deg_pass(e_hbm, out_hbm, idxv, ones_v, zbuf, acc_sh):
    cid = lax.axis_index("c")
    sid = lax.axis_index("s")
    wid = cid * NS + sid

    @pl.loop(0, CH)
    def _(i):
        ones_v[i, :] = jnp.ones((16,), jnp.float32)

    _zero_fill(zbuf, DEGW)
    _zero_shared(acc_sh, zbuf, sid)
    plsc.subcore_barrier()

    base = wid * NCHUNK

    @pl.loop(0, NCHUNK)
    def _(j):
        pltpu.sync_copy(e_hbm.at[base + j], idxv)
        pltpu.sync_copy(ones_v, acc_sh.at[idxv.at[1]], add=True)

    plsc.subcore_barrier()
    pltpu.sync_copy(acc_sh.at[pl.ds(sid * RPS, RPS)],
                    out_hbm.at[cid, pl.ds(sid * RPS, RPS)])


_RING = 4


def _make_edge_pass(f):
    h = f // 2
    ngroup = NCHUNK2 // _RING

    @functools.partial(
        pl.kernel,
        out_type=jax.ShapeDtypeStruct((NC, NP, h), jnp.float32),
        mesh=_mesh(),
        compiler_params=_sc_params,
        scratch_types=[
            [pltpu.VMEM((2, CH), jnp.int32) for _ in range(_RING)],
            [pltpu.VMEM((CH, h), jnp.float32) for _ in range(_RING)],
            pltpu.VMEM((64, h), jnp.float32),
            pltpu.VMEM_SHARED((NP, h), jnp.float32),
            pltpu.VMEM_SHARED((NP, h), jnp.float32),
            [pltpu.SemaphoreType.DMA for _ in range(_RING)],
            [pltpu.SemaphoreType.DMA for _ in range(_RING)],
        ],
    )
    def edge_pass(hs_hbm, e_hbm, out_hbm,
                  idx, rows, zbuf, tab_sh, acc_sh, gsem, ssem):
        cid = lax.axis_index("c")
        sid = lax.axis_index("s")

        pltpu.sync_copy(hs_hbm.at[cid, pl.ds(sid * RPS, RPS)],
                        tab_sh.at[pl.ds(sid * RPS, RPS)])
        _zero_fill(zbuf, h)
        _zero_shared(acc_sh, zbuf, sid)
        plsc.subcore_barrier()

        base = sid * NCHUNK2

        def wait_gather(r):
            pltpu.make_async_copy(hs_hbm.at[0, pl.ds(0, CH)],
                                  rows[r], gsem[r]).wait()

        def wait_scatter(r):
            pltpu.make_async_copy(rows[r], acc_sh.at[pl.ds(0, CH)],
                                  ssem[r]).wait()

        @pl.loop(0, ngroup)
        def _(p):
            g0 = base + _RING * p
            for r in range(_RING):
                @pl.when(p > 0)
                def _(r=r):
                    wait_scatter(r)
                pltpu.sync_copy(e_hbm.at[g0 + r], idx[r])
                pltpu.async_copy(tab_sh.at[idx[r].at[0]], rows[r], gsem[r])
            for r in range(_RING):
                wait_gather(r)
                pltpu.async_copy(rows[r], acc_sh.at[idx[r].at[1]],
                                 ssem[r], add=True)

        for r in range(_RING):
            wait_scatter(r)
        plsc.subcore_barrier()
        pltpu.sync_copy(acc_sh.at[pl.ds(sid * RPS, RPS)],
                        out_hbm.at[cid, pl.ds(sid * RPS, RPS)])

    return edge_pass


_edge_pass_h = _make_edge_pass(DH)
_edge_pass_o = _make_edge_pass(DO)


def _dis_from_deg(deg_ref):
    deg = deg_ref[0, :, 0:1] + deg_ref[1, :, 0:1]
    rows = lax.broadcasted_iota(jnp.int32, (NP, 1), 0)
    deg = deg + jnp.where(rows < N, 1.0, 0.0)
    return jnp.where(deg > 0, lax.rsqrt(deg), 0.0)


def _tc1_body(x_ref, w_ref, deg_ref, o_ref):
    dis = _dis_from_deg(deg_ref)
    hh = jnp.dot(x_ref[...], w_ref[...],
                 preferred_element_type=jnp.float32) * dis
    o_ref[0] = hh[:, :DH // 2]
    o_ref[1] = hh[:, DH // 2:]


def _tc2_body(acc_ref, hs1_ref, deg_ref, b1_ref, g_ref, be_ref, w_ref, o_ref):
    dis = _dis_from_deg(deg_ref)
    acc = jnp.concatenate([acc_ref[0] + hs1_ref[0],
                           acc_ref[1] + hs1_ref[1]], axis=1)
    t = acc * dis + b1_ref[...]
    t = jnp.maximum(t, 0.0)
    mu = jnp.mean(t, axis=-1, keepdims=True)
    var = jnp.mean((t - mu) ** 2, axis=-1, keepdims=True)
    h = (t - mu) * lax.rsqrt(var + 1e-5) * g_ref[...] + be_ref[...]
    hs2 = jnp.dot(h, w_ref[...], preferred_element_type=jnp.float32) * dis
    o_ref[0] = hs2[:, :DO // 2]
    o_ref[1] = hs2[:, DO // 2:]


def _tc3_body(acc_ref, hs2_ref, deg_ref, b2_ref, o_ref):
    dis = _dis_from_deg(deg_ref)
    res = jnp.concatenate([acc_ref[0] + hs2_ref[0],
                           acc_ref[1] + hs2_ref[1]], axis=1)
    o_ref[...] = (res * dis + b2_ref[...])[:N]


_tc1 = pl.pallas_call(_tc1_body,
                      out_shape=jax.ShapeDtypeStruct((NC, NP, DH // 2),
                                                     jnp.float32))
_tc2 = pl.pallas_call(_tc2_body,
                      out_shape=jax.ShapeDtypeStruct((NC, NP, DO // 2),
                                                     jnp.float32))
_tc3 = pl.pallas_call(_tc3_body,
                      out_shape=jax.ShapeDtypeStruct((N, DO), jnp.float32))


def kernel(x, edge_index, W1, b1, gamma, beta, W2, b2):
    ei = edge_index.astype(jnp.int32)
    pad = jnp.full((2, EP - E), N, jnp.int32)
    e3 = jnp.concatenate([ei, pad], axis=1)
    e3 = e3.reshape(2, NT * NCHUNK, CH).transpose(1, 0, 2)
    x_pad = jnp.zeros((NP, D_IN), jnp.float32).at[:N].set(x)

    degp = _deg_pass(e3)
    hs1 = _tc1(x_pad, W1, degp)
    acc1 = _edge_pass_h(hs1, e3)
    hs2 = _tc2(acc1, hs1, degp, b1.reshape(1, DH),
               gamma.reshape(1, DH), beta.reshape(1, DH), W2)
    acc2 = _edge_pass_o(hs2, e3)
    return _tc3(acc2, hs2, degp, b2.reshape(1, DO))
